# baseline (speedup 1.0000x reference)
import functools
import jax
import jax.numpy as jnp
from jax import lax
from jax.experimental import pallas as pl
from jax.experimental.pallas import tpu as pltpu
from jax.experimental.pallas import tpu_sc as plsc

_N_SC = 2
_N_SUB = 16
_LANES = 16


def _actglu(x, W1, b1, W2, b2):
    h = x @ W1 + b1
    half = h.shape[-1] // 2
    a = h[:, :half]
    g = h[:, half:]
    return (a * jax.nn.relu(g)) @ W2 + b2


def _full_spec(a):
    return pl.BlockSpec(a.shape, lambda i: tuple(0 for _ in a.shape))


def _edge_body(e_ref, W11, b11, W12, b12, W21, b21, W22, b22,
               Wek, bek, Wev, bev, al,
               eout_ref, ek_ref, ev_ref):
    ae = al[0, 0]
    e = e_ref[...]
    e1 = e + ae * _actglu(e, W11[...], b11[...], W12[...], b12[...])
    ek_ref[...] = e1 @ Wek[...] + bek[...]
    ev_ref[...] = e1 @ Wev[...] + bev[...]
    eout_ref[...] = e1 + ae * _actglu(e1, W21[...], b21[...], W22[...], b22[...])


def _edge_stage(e, ef1_W1, ef1_b1, ef1_W2, ef1_b2,
                ef2_W1, ef2_b1, ef2_W2, ef2_b2,
                Wek, bek, Wev, bev, alpha_e, block=2000):
    E, D = e.shape
    HK = Wek.shape[1]
    grid = (E + block - 1) // block
    row_spec = pl.BlockSpec((block, D), lambda i: (i, 0))
    out_spec = pl.BlockSpec((block, HK), lambda i: (i, 0))
    al = alpha_e.reshape(1, 1)
    ws = [ef1_W1, ef1_b1.reshape(1, -1), ef1_W2, ef1_b2.reshape(1, -1),
          ef2_W1, ef2_b1.reshape(1, -1), ef2_W2, ef2_b2.reshape(1, -1),
          Wek, bek.reshape(1, -1), Wev, bev.reshape(1, -1), al]
    return pl.pallas_call(
        _edge_body,
        grid=(grid,),
        in_specs=[row_spec] + [_full_spec(w) for w in ws],
        out_specs=[row_spec, out_spec, out_spec],
        out_shape=[
            jax.ShapeDtypeStruct((E, D), jnp.float32),
            jax.ShapeDtypeStruct((E, HK), jnp.float32),
            jax.ShapeDtypeStruct((E, HK), jnp.float32),
        ],
    )(e, *ws)


def _node_body(n_ref, W11, b11, W12, b12, Wq, bq, Wk, bk, Wv, bv, al,
               n1_ref, q_ref, k_ref, v_ref):
    an = al[0, 0]
    n = n_ref[...]
    n1 = n + an * _actglu(n, W11[...], b11[...], W12[...], b12[...])
    n1_ref[...] = n1
    q_ref[...] = n1 @ Wq[...] + bq[...]
    k_ref[...] = n1 @ Wk[...] + bk[...]
    v_ref[...] = n1 @ Wv[...] + bv[...]


def _node_stage(n, nf1_W1, nf1_b1, nf1_W2, nf1_b2,
                Wq, bq, Wk, bk, Wv, bv, alpha_n, block=1000):
    N, D = n.shape
    HK = Wq.shape[1]
    grid = (N + block - 1) // block
    row_spec = pl.BlockSpec((block, D), lambda i: (i, 0))
    out_spec = pl.BlockSpec((block, HK), lambda i: (i, 0))
    al = alpha_n.reshape(1, 1)
    ws = [nf1_W1, nf1_b1.reshape(1, -1), nf1_W2, nf1_b2.reshape(1, -1),
          Wq, bq.reshape(1, -1), Wk, bk.reshape(1, -1), Wv, bv.reshape(1, -1), al]
    return pl.pallas_call(
        _node_body,
        grid=(grid,),
        in_specs=[row_spec] + [_full_spec(w) for w in ws],
        out_specs=[row_spec, out_spec, out_spec, out_spec],
        out_shape=[
            jax.ShapeDtypeStruct((N, D), jnp.float32),
            jax.ShapeDtypeStruct((N, HK), jnp.float32),
            jax.ShapeDtypeStruct((N, HK), jnp.float32),
            jax.ShapeDtypeStruct((N, HK), jnp.float32),
        ],
    )(n, *ws)


def _att_body(qd_ref, ks_ref, ekc_ref, M_ref, ex_ref):
    qd = qd_ref[...]
    M = M_ref[...]
    s1 = (qd * ks_ref[...]) @ M
    s2 = (qd * ekc_ref[...]) @ M
    ex_ref[...] = jnp.exp(jnp.concatenate([s1, s2], axis=-1))


def _att_stage(qd, ks, ek, M, block=2000):
    E, HK = qd.shape
    grid = (E + block - 1) // block
    row_spec = pl.BlockSpec((block, HK), lambda i: (i, 0))
    ex_spec = pl.BlockSpec((block, 16), lambda i: (i, 0))
    return pl.pallas_call(
        _att_body,
        grid=(grid,),
        in_specs=[row_spec, row_spec, row_spec, _full_spec(M)],
        out_specs=ex_spec,
        out_shape=jax.ShapeDtypeStruct((E, 16), jnp.float32),
    )(qd, ks, ek, M)


def _w_body(vs_ref, ev_ref, ex_ref, dd_ref, MT_ref, ct_ref):
    ex = ex_ref[...]
    dd = dd_ref[...]
    MT = MT_ref[...]
    d1 = dd[:, :8] + dd[:, 16:24]
    d2 = dd[:, 8:16] + dd[:, 24:32]
    a1 = ex[:, :8] / (d1 + 1e-9)
    a2 = ex[:, 8:] / (d2 + 1e-9)
    ct_ref[...] = ((a1 + a2) @ MT) * vs_ref[...] + (a2 @ MT) * ev_ref[...]


def _w_stage(vs, ev, ex, dd32, MT, block=2000):
    E, HK = vs.shape
    grid = (E + block - 1) // block
    row_spec = pl.BlockSpec((block, HK), lambda i: (i, 0))
    ex_spec = pl.BlockSpec((block, 16), lambda i: (i, 0))
    dd_spec = pl.BlockSpec((block, 32), lambda i: (i, 0))
    return pl.pallas_call(
        _w_body,
        grid=(grid,),
        in_specs=[row_spec, row_spec, ex_spec, dd_spec, _full_spec(MT)],
        out_specs=row_spec,
        out_shape=jax.ShapeDtypeStruct((E, HK), jnp.float32),
    )(vs, ev, ex, dd32, MT)


def _final_body(z0_ref, z1_ref, z2_ref, z3_ref, n1_ref, ln_g, ln_b, Wm, bm,
                W21, b21, W22, b22, al, nout_ref):
    an = al[0, 0]
    z = jnp.concatenate(
        [z0_ref[...], z1_ref[...], z2_ref[...], z3_ref[...]], axis=-1)
    mu = jnp.mean(z, axis=-1, keepdims=True)
    var = jnp.mean(jnp.square(z - mu), axis=-1, keepdims=True)
    zn = (z - mu) / jnp.sqrt(var + 1e-5) * ln_g[...] + ln_b[...]
    mix = jax.nn.relu(zn @ Wm[...] + bm[...])
    n2 = n1_ref[...] + an * mix
    nout_ref[...] = n2 + an * _actglu(n2, W21[...], b21[...], W22[...], b22[...])


def _final_stage(z4, n1, ln_g, ln_b, Wm, bm,
                 nf2_W1, nf2_b1, nf2_W2, nf2_b2, alpha_n, block=1000):
    N, D = n1.shape
    grid = (N + block - 1) // block
    zp_spec = pl.BlockSpec((block, 128), lambda i: (i, 0))
    row_spec = pl.BlockSpec((block, D), lambda i: (i, 0))
    al = alpha_n.reshape(1, 1)
    ws = [ln_g.reshape(1, -1), ln_b.reshape(1, -1), Wm, bm.reshape(1, -1),
          nf2_W1, nf2_b1.reshape(1, -1), nf2_W2, nf2_b2.reshape(1, -1), al]
    return pl.pallas_call(
        _final_body,
        grid=(grid,),
        in_specs=[zp_spec] * 4 + [row_spec] + [_full_spec(w) for w in ws],
        out_specs=row_spec,
        out_shape=jax.ShapeDtypeStruct((N, D), jnp.float32),
    )(z4[0], z4[1], z4[2], z4[3], n1, *ws)


_SC_PARAMS = dict(
    compiler_params=pltpu.CompilerParams(
        use_tc_tiling_on_sc=False, needs_layout_passes=False))
_MESH = dict(core_axis_name="c", subcore_axis_name="s")


def _gather_qkv(q, k, v, eit):
    N, HK = q.shape
    E = eit.shape[0]
    C = 32
    NCHUNK = E // C
    NW = _N_SC * _N_SUB
    HALF = ((NCHUNK + NW - 1) // NW + 1) // 2

    @functools.partial(
        pl.kernel, mesh=plsc.VectorSubcoreMesh(**_MESH), **_SC_PARAMS,
        out_type=[jax.ShapeDtypeStruct((E, HK), jnp.float32)] * 3,
        scratch_types=[
            pltpu.VMEM((C, 2), jnp.int32), pltpu.VMEM((C, 2), jnp.int32),
            pltpu.VMEM((C,), jnp.int32), pltpu.VMEM((C,), jnp.int32),
            pltpu.VMEM((C,), jnp.int32), pltpu.VMEM((C,), jnp.int32),
            pltpu.VMEM((C, 512), jnp.float32), pltpu.VMEM((C, 512), jnp.float32),
            pltpu.VMEM((C, 512), jnp.float32), pltpu.VMEM((C, 512), jnp.float32),
            pltpu.VMEM((C, 512), jnp.float32), pltpu.VMEM((C, 512), jnp.float32),
            pltpu.SemaphoreType.DMA, pltpu.SemaphoreType.DMA,
            pltpu.SemaphoreType.DMA, pltpu.SemaphoreType.DMA,
            pltpu.SemaphoreType.DMA, pltpu.SemaphoreType.DMA,
        ])
    def g1(q_hbm, k_hbm, v_hbm, eit_hbm, qd_hbm, ks_hbm, vs_hbm,
           eit0, eit1, src0, src1, dst0, dst1,
           qd0, qd1, ks0, ks1, vs0, vs1,
           sq0, sq1, sk0, sk1, sv0, sv1):
        eitb = [eit0, eit1]
        srcb = [src0, src1]
        dstb = [dst0, dst1]
        qdb = [qd0, qd1]
        ksb = [ks0, ks1]
        vsb = [vs0, vs1]
        sq = [sq0, sq1]
        sk = [sk0, sk1]
        sv = [sv0, sv1]
        c = lax.axis_index("c")
        s = lax.axis_index("s")
        wid = s * _N_SC + c
        iota = lax.iota(jnp.int32, _LANES)
        zero16 = jnp.zeros((_LANES,), jnp.int32)
        one16 = jnp.full((_LANES,), 1, jnp.int32)

        def issue(chunk, b):
            base = chunk * C
            pltpu.sync_copy(eit_hbm.at[pl.ds(base, C)], eitb[b])
            for g in range(C // _LANES):
                rows = g * _LANES + iota
                sl = pl.ds(g * _LANES, _LANES)
                srcb[b][sl] = plsc.load_gather(eitb[b], [rows, zero16])
                dstb[b][sl] = plsc.load_gather(eitb[b], [rows, one16])
            pltpu.async_copy(q_hbm.at[dstb[b]], qdb[b], sq[b])
            pltpu.async_copy(k_hbm.at[srcb[b]], ksb[b], sk[b])
            pltpu.async_copy(v_hbm.at[srcb[b]], vsb[b], sv[b])

        issue(wid, 0)

        def body(j2, carry):
            for b in range(2):
                j = j2 * 2 + b
                chunk = wid + NW * j
                chunk_n = wid + NW * (j + 1)

                @pl.when(chunk_n < NCHUNK)
                def _():
                    issue(chunk_n, 1 - b)

                @pl.when(chunk < NCHUNK)
                def _():
                    base = chunk * C
                    pltpu.make_async_copy(q_hbm.at[dstb[b]], qdb[b],
                                          sq[b]).wait()
                    pltpu.make_async_copy(k_hbm.at[srcb[b]], ksb[b],
                                          sk[b]).wait()
                    pltpu.make_async_copy(v_hbm.at[srcb[b]], vsb[b],
                                          sv[b]).wait()
                    pltpu.sync_copy(qdb[b], qd_hbm.at[pl.ds(base, C)])
                    pltpu.sync_copy(ksb[b], ks_hbm.at[pl.ds(base, C)])
                    pltpu.sync_copy(vsb[b], vs_hbm.at[pl.ds(base, C)])
            return carry

        lax.fori_loop(0, HALF, body, 0)

    return g1(q, k, v, eit)


def _scatter_den(ex, eit, zeros):
    N = zeros.shape[0]
    E = ex.shape[0]
    C = 64
    NCHUNK = E // C
    NW = _N_SC * _N_SUB
    HALF = ((NCHUNK + NW - 1) // NW + 1) // 2
    STRIPE = (N // (8 * _N_SUB)) * 8
    TAIL = N - STRIPE * _N_SUB

    @functools.partial(
        pl.kernel, mesh=plsc.VectorSubcoreMesh(**_MESH), **_SC_PARAMS,
        out_type=jax.ShapeDtypeStruct((_N_SC, N, 16), jnp.float32),
        scratch_types=[
            pltpu.VMEM((C, 2), jnp.int32), pltpu.VMEM((C, 2), jnp.int32),
            pltpu.VMEM((C,), jnp.int32), pltpu.VMEM((C,), jnp.int32),
            pltpu.VMEM((C, 16), jnp.float32), pltpu.VMEM((C, 16), jnp.float32),
            pltpu.VMEM_SHARED((N, 16), jnp.float32),
            pltpu.SemaphoreType.DMA, pltpu.SemaphoreType.DMA,
        ])
    def g2(ex_hbm, eit_hbm, z_hbm, den_hbm,
           eit0, eit1, dst0, dst1, ex0, ex1, den_sh, se0, se1):
        eitb = [eit0, eit1]
        dstb = [dst0, dst1]
        exb = [ex0, ex1]
        se = [se0, se1]
        c = lax.axis_index("c")
        s = lax.axis_index("s")
        wid = s * _N_SC + c
        iota = lax.iota(jnp.int32, _LANES)
        one16 = jnp.full((_LANES,), 1, jnp.int32)
        pltpu.sync_copy(z_hbm.at[pl.ds(s * STRIPE, STRIPE)],
                        den_sh.at[pl.ds(s * STRIPE, STRIPE)])

        @pl.when(s == 0)
        def _():
            pltpu.sync_copy(z_hbm.at[pl.ds(STRIPE * _N_SUB, TAIL)],
                            den_sh.at[pl.ds(STRIPE * _N_SUB, TAIL)])

        plsc.subcore_barrier()

        def issue(chunk, b):
            base = chunk * C
            pltpu.sync_copy(eit_hbm.at[pl.ds(base, C)], eitb[b])
            for g in range(C // _LANES):
                rows = g * _LANES + iota
                dstb[b][pl.ds(g * _LANES, _LANES)] = plsc.load_gather(
                    eitb[b], [rows, one16])
            pltpu.async_copy(ex_hbm.at[pl.ds(base, C)], exb[b], se[b])

        issue(wid, 0)

        def body(j2, carry):
            for b in range(2):
                j = j2 * 2 + b
                chunk = wid + NW * j
                chunk_n = wid + NW * (j + 1)

                @pl.when(chunk_n < NCHUNK)
                def _():
                    issue(chunk_n, 1 - b)

                @pl.when(chunk < NCHUNK)
                def _():
                    base = chunk * C
                    pltpu.make_async_copy(ex_hbm.at[pl.ds(base, C)],
                                          exb[b], se[b]).wait()
                    pltpu.sync_copy(exb[b], den_sh.at[dstb[b]], add=True)
            return carry

        lax.fori_loop(0, HALF, body, 0)
        plsc.subcore_barrier()
        pltpu.sync_copy(den_sh.at[pl.ds(s * STRIPE, STRIPE)],
                        den_hbm.at[c, pl.ds(s * STRIPE, STRIPE)])

        @pl.when(s == 0)
        def _():
            pltpu.sync_copy(den_sh.at[pl.ds(STRIPE * _N_SUB, TAIL)],
                            den_hbm.at[c, pl.ds(STRIPE * _N_SUB, TAIL)])

    return g2(ex, eit, zeros)


def _gather_den(dennm, eit):
    N = dennm.shape[0]
    E = eit.shape[0]
    C = 64
    NCHUNK = E // C
    NW = _N_SC * _N_SUB
    HALF = ((NCHUNK + NW - 1) // NW + 1) // 2

    @functools.partial(
        pl.kernel, mesh=plsc.VectorSubcoreMesh(**_MESH), **_SC_PARAMS,
        out_type=jax.ShapeDtypeStruct((E, 2, 16), jnp.float32),
        scratch_types=[
            pltpu.VMEM((C, 2), jnp.int32), pltpu.VMEM((C, 2), jnp.int32),
            pltpu.VMEM((C,), jnp.int32), pltpu.VMEM((C,), jnp.int32),
            pltpu.VMEM((C, 2, 16), jnp.float32),
            pltpu.VMEM((C, 2, 16), jnp.float32),
            pltpu.SemaphoreType.DMA, pltpu.SemaphoreType.DMA,
        ])
    def g2b(den_hbm, eit_hbm, dend_hbm,
            eit0, eit1, dst0, dst1, dd0, dd1, sd0, sd1):
        eitb = [eit0, eit1]
        dstb = [dst0, dst1]
        ddb = [dd0, dd1]
        sd = [sd0, sd1]
        c = lax.axis_index("c")
        s = lax.axis_index("s")
        wid = s * _N_SC + c
        iota = lax.iota(jnp.int32, _LANES)
        one16 = jnp.full((_LANES,), 1, jnp.int32)

        def issue(chunk, b):
            base = chunk * C
            pltpu.sync_copy(eit_hbm.at[pl.ds(base, C)], eitb[b])
            for g in range(C // _LANES):
                rows = g * _LANES + iota
                dstb[b][pl.ds(g * _LANES, _LANES)] = plsc.load_gather(
                    eitb[b], [rows, one16])
            pltpu.async_copy(den_hbm.at[dstb[b]], ddb[b], sd[b])

        issue(wid, 0)

        def body(j2, carry):
            for b in range(2):
                j = j2 * 2 + b
                chunk = wid + NW * j
                chunk_n = wid + NW * (j + 1)

                @pl.when(chunk_n < NCHUNK)
                def _():
                    issue(chunk_n, 1 - b)

                @pl.when(chunk < NCHUNK)
                def _():
                    base = chunk * C
                    pltpu.make_async_copy(den_hbm.at[dstb[b]], ddb[b],
                                          sd[b]).wait()
                    pltpu.sync_copy(ddb[b], dend_hbm.at[pl.ds(base, C)])
            return carry

        lax.fori_loop(0, HALF, body, 0)

    return g2b(dennm, eit)


def _scatter_z(contrib, eit, zeros):
    N = zeros.shape[0]
    E = contrib.shape[0]
    C = 64
    NCHUNK = E // C
    HALF = ((NCHUNK + _N_SUB - 1) // _N_SUB + 1) // 2
    STRIPE = (N // (8 * _N_SUB)) * 8
    TAIL = N - STRIPE * _N_SUB

    @functools.partial(
        pl.kernel, mesh=plsc.VectorSubcoreMesh(**_MESH), **_SC_PARAMS,
        out_type=jax.ShapeDtypeStruct((4, N, 128), jnp.float32),
        scratch_types=[
            pltpu.VMEM((C, 2), jnp.int32), pltpu.VMEM((C, 2), jnp.int32),
            pltpu.VMEM((C,), jnp.int32), pltpu.VMEM((C,), jnp.int32),
            pltpu.VMEM((C, 128), jnp.float32), pltpu.VMEM((C, 128), jnp.float32),
            pltpu.VMEM_SHARED((N, 128), jnp.float32),
            pltpu.SemaphoreType.DMA, pltpu.SemaphoreType.DMA,
        ])
    def g3(ct_hbm, eit_hbm, z_hbm, zout_hbm,
           eit0, eit1, dst0, dst1, ct0, ct1, z_sh, sc0, sc1):
        eitb = [eit0, eit1]
        dstb = [dst0, dst1]
        ctb = [ct0, ct1]
        sc = [sc0, sc1]
        c = lax.axis_index("c")
        s = lax.axis_index("s")
        iota = lax.iota(jnp.int32, _LANES)
        one16 = jnp.full((_LANES,), 1, jnp.int32)

        for r in range(2):
            pr = 2 * r + c

            def issue(chunk, b, pr=pr):
                base = chunk * C
                pltpu.sync_copy(eit_hbm.at[pl.ds(base, C)], eitb[b])
                for g in range(C // _LANES):
                    rows = g * _LANES + iota
                    dstb[b][pl.ds(g * _LANES, _LANES)] = plsc.load_gather(
                        eitb[b], [rows, one16])
                pltpu.async_copy(
                    ct_hbm.at[pl.ds(base, C), pl.ds(pr * 128, 128)],
                    ctb[b], sc[b])

            pltpu.sync_copy(z_hbm.at[pl.ds(s * STRIPE, STRIPE)],
                            z_sh.at[pl.ds(s * STRIPE, STRIPE)])

            @pl.when(s == 0)
            def _():
                pltpu.sync_copy(z_hbm.at[pl.ds(STRIPE * _N_SUB, TAIL)],
                                z_sh.at[pl.ds(STRIPE * _N_SUB, TAIL)])

            plsc.subcore_barrier()
            issue(s, 0)

            def body(j2, carry):
                for b in range(2):
                    j = j2 * 2 + b
                    chunk = s + _N_SUB * j
                    chunk_n = s + _N_SUB * (j + 1)

                    @pl.when(chunk_n < NCHUNK)
                    def _():
                        issue(chunk_n, 1 - b)

                    @pl.when(chunk < NCHUNK)
                    def _():
                        base = chunk * C
                        pltpu.make_async_copy(
                            ct_hbm.at[pl.ds(base, C), pl.ds(pr * 128, 128)],
                            ctb[b], sc[b]).wait()
                        pltpu.sync_copy(ctb[b], z_sh.at[dstb[b]], add=True)
                return carry

            lax.fori_loop(0, HALF, body, 0)
            plsc.subcore_barrier()
            pltpu.sync_copy(z_sh.at[pl.ds(s * STRIPE, STRIPE)],
                            zout_hbm.at[pr, pl.ds(s * STRIPE, STRIPE)])

            @pl.when(s == 0)
            def _():
                pltpu.sync_copy(z_sh.at[pl.ds(STRIPE * _N_SUB, TAIL)],
                                zout_hbm.at[pr, pl.ds(STRIPE * _N_SUB, TAIL)])

    return g3(contrib, eit, zeros)


def kernel(n, e, edge_index,
           nf1_W1, nf1_b1, nf1_W2, nf1_b2,
           ef1_W1, ef1_b1, ef1_W2, ef1_b2,
           nf2_W1, nf2_b1, nf2_W2, nf2_b2,
           ef2_W1, ef2_b1, ef2_W2, ef2_b2,
           Wq, Wk, Wv, Weq, Wek, Wev,
           bq, bk, bv, beq, bek, bev,
           ln_g, ln_b, Wm, bm, alpha_n, alpha_e):
    N, D = n.shape
    E = e.shape[0]

    e_out, ek, ev = _edge_stage(
        e, ef1_W1, ef1_b1, ef1_W2, ef1_b2,
        ef2_W1, ef2_b1, ef2_W2, ef2_b2,
        Wek, bek, Wev, bev, alpha_e)

    n1, q, k, v = _node_stage(
        n, nf1_W1, nf1_b1, nf1_W2, nf1_b2,
        Wq, bq, Wk, bk, Wv, bv, alpha_n)

    eit = edge_index.T.reshape(E, 2)
    qd, ks, vs = _gather_qkv(q, k, v, eit)

    M = (jnp.arange(512)[:, None] // 64 ==
         jnp.arange(8)[None, :]).astype(jnp.float32)
    ex = _att_stage(qd, ks, ek, M)

    zeros16 = jnp.zeros((N, 16), jnp.float32)
    den = _scatter_den(ex, eit, zeros16)
    dennm = den.transpose(1, 0, 2).reshape(N, 2, 16)
    dend = _gather_den(dennm, eit)

    contrib = _w_stage(vs, ev, ex, dend.reshape(E, 32), M.T)

    zeros128 = jnp.zeros((N, 128), jnp.float32)
    z4 = _scatter_z(contrib, eit, zeros128)

    n_out = _final_stage(z4, n1, ln_g, ln_b, Wm, bm,
                         nf2_W1, nf2_b1, nf2_W2, nf2_b2, alpha_n)
    return n_out, e_out
